# baseline (speedup 1.0000x reference)
import functools

import jax
import jax.numpy as jnp
from jax import lax
from jax.experimental import pallas as pl
from jax.experimental.pallas import tpu as pltpu
from jax.experimental.pallas import tpu_sc as plsc

N_ROWS = 20000
N_COLS = 512
N_OUT = 4
NC = 2
NS = 16
NW = NC * NS
L = 16

ROWS_MAIN = 624
BLOCK_ROWS = 48
N_BLOCKS = ROWS_MAIN // BLOCK_ROWS
N_CHUNKS = N_COLS // L
N_ACC = 4
CPB = N_CHUNKS // N_ACC

MATCHED_T = 0.5
UNMATCHED_T = 0.4


def _scan_row(buf, rr, iota16):
    ninf = jnp.full((L,), -jnp.inf, jnp.float32)
    ms = [ninf for _ in range(N_ACC)]
    ts = [jnp.zeros((L,), jnp.int32) for _ in range(N_ACC)]
    tvec = jnp.zeros((L,), jnp.int32)
    for t in range(CPB):
        for a in range(N_ACC):
            k = a * CPB + t
            v = buf[rr, pl.ds(k * L, L)]
            cond = v > ms[a]
            ms[a] = jnp.maximum(ms[a], v)
            ts[a] = jnp.where(cond, tvec, ts[a])
        tvec = tvec + 1
    m, tw = ms[0], ts[0]
    aw = jnp.zeros((L,), jnp.int32)
    for a in range(1, N_ACC):
        cond = ms[a] > m
        m = jnp.maximum(m, ms[a])
        tw = jnp.where(cond, ts[a], tw)
        aw = jnp.where(cond, jnp.full((L,), a, jnp.int32), aw)
    colv = (aw * CPB + tw) * L + iota16
    rmax = jnp.max(m)
    cand = jnp.where(m == rmax, colv, jnp.full((L,), N_COLS, jnp.int32))
    argc = jnp.min(cand)
    return rmax, argc


def _process_group(buf, g, out_v, orow0, mv_v, uvec, ivec, iota16):
    def row_body(i, carry):
        resm, resc = carry
        for u in range(2):
            r = 2 * i + u
            rmax, argc = _scan_row(buf, g * L + r, iota16)
            lanemask = iota16 == r
            resm = jnp.where(lanemask, rmax, resm)
            resc = jnp.where(lanemask, argc, resc)
        return resm, resc

    resm = jnp.full((L,), -jnp.inf, jnp.float32)
    resc = jnp.zeros((L,), jnp.int32)
    resm, resc = lax.fori_loop(0, L // 2, row_body, (resm, resc))

    below = UNMATCHED_T > resm
    between = jnp.logical_and(resm >= UNMATCHED_T, MATCHED_T > resm)
    c4 = resc * N_OUT
    orow4 = orow0 * N_OUT + iota16 * N_OUT
    for j in range(N_OUT):
        gj = plsc.load_gather(mv_v, [c4 + j])
        o = jnp.where(below, uvec, gj)
        o = jnp.where(between, ivec, o)
        plsc.store_scatter(out_v, [orow4 + j], o)


def _body(sim, mv, unm, ign, out, buf0, buf1, mv_v, scal_v, out_v, ebuf,
          eout_v, sem0, sem1):
    c = lax.axis_index("c")
    s = lax.axis_index("s")
    wid = s * NC + c

    pltpu.sync_copy(mv, mv_v)
    pltpu.sync_copy(unm, scal_v.at[pl.ds(0, 1)])
    pltpu.sync_copy(ign, scal_v.at[pl.ds(8, 1)])
    uvec = plsc.load_gather(scal_v, [jnp.zeros((L,), jnp.int32)])
    ivec = plsc.load_gather(scal_v, [jnp.full((L,), 8, jnp.int32)])
    iota16 = lax.iota(jnp.int32, L)

    row0 = wid * ROWS_MAIN

    def src(blk):
        return sim.at[pl.ds(row0 + blk * BLOCK_ROWS, BLOCK_ROWS), :]

    def proc_block(buf, blk):
        for g in range(BLOCK_ROWS // L):
            _process_group(buf, g, out_v, blk * BLOCK_ROWS + g * L,
                           mv_v, uvec, ivec, iota16)

    pltpu.async_copy(src(0), buf0, sem0)

    def pair_body(t, carry):
        blk = 2 * t
        pltpu.async_copy(src(blk + 1), buf1, sem1)
        pltpu.make_async_copy(src(blk), buf0, sem0).wait()
        proc_block(buf0, blk)
        pltpu.async_copy(src(blk + 2), buf0, sem0)
        pltpu.make_async_copy(src(blk + 1), buf1, sem1).wait()
        proc_block(buf1, blk + 1)
        return carry

    lax.fori_loop(0, (N_BLOCKS - 1) // 2, pair_body, 0)
    pltpu.make_async_copy(src(N_BLOCKS - 1), buf0, sem0).wait()
    proc_block(buf0, N_BLOCKS - 1)

    pltpu.sync_copy(out_v, out.at[pl.ds(row0 * N_OUT, ROWS_MAIN * N_OUT)])

    @pl.when(wid < 2)
    def _extra():
        er0 = NW * ROWS_MAIN + wid * L
        pltpu.sync_copy(sim.at[pl.ds(er0, L), :], ebuf)
        _process_group(ebuf, 0, eout_v, 0, mv_v, uvec, ivec, iota16)
        pltpu.sync_copy(eout_v, out.at[pl.ds(er0 * N_OUT, L * N_OUT)])


_matcher = functools.partial(
    pl.kernel,
    out_type=jax.ShapeDtypeStruct((N_ROWS * N_OUT,), jnp.float32),
    mesh=plsc.VectorSubcoreMesh(core_axis_name="c", subcore_axis_name="s"),
    compiler_params=pltpu.CompilerParams(needs_layout_passes=False),
    scratch_types=[
        pltpu.VMEM((BLOCK_ROWS, N_COLS), jnp.float32),
        pltpu.VMEM((BLOCK_ROWS, N_COLS), jnp.float32),
        pltpu.VMEM((N_COLS * N_OUT,), jnp.float32),
        pltpu.VMEM((L,), jnp.float32),
        pltpu.VMEM((ROWS_MAIN * N_OUT,), jnp.float32),
        pltpu.VMEM((L, N_COLS), jnp.float32),
        pltpu.VMEM((L * N_OUT,), jnp.float32),
        pltpu.SemaphoreType.DMA,
        pltpu.SemaphoreType.DMA,
    ],
)(_body)


def kernel(similarity, matched_values, unmatched_values, ignored_values):
    out = _matcher(similarity, matched_values.reshape(-1), unmatched_values,
                   ignored_values)
    return out.reshape(N_ROWS, N_OUT)


# speedup vs baseline: 1.3445x; 1.3445x over previous
import functools

import jax
import jax.numpy as jnp
from jax import lax
from jax.experimental import pallas as pl
from jax.experimental.pallas import tpu as pltpu
from jax.experimental.pallas import tpu_sc as plsc

N_ROWS = 20000
N_COLS = 512
N_OUT = 4
NC = 2
NS = 16
NW = NC * NS
L = 16

SLAB = 640
OUT_ROWS = 8
PAD_COLS = NW * SLAB
BLOCK_ROWS = 48
FULL_BLOCKS = 13
TAIL_BLOCKS = 3
N_CHUNKS = N_COLS // L
N_ACC = 4
CPB = N_CHUNKS // N_ACC

MATCHED_T = 0.5
UNMATCHED_T = 0.4


def _scan_row(buf, rr, iota16):
    ninf = jnp.full((L,), -jnp.inf, jnp.float32)
    ms = [ninf for _ in range(N_ACC)]
    ts = [jnp.zeros((L,), jnp.int32) for _ in range(N_ACC)]
    tvec = jnp.zeros((L,), jnp.int32)
    for t in range(CPB):
        for a in range(N_ACC):
            k = a * CPB + t
            v = buf[rr, pl.ds(k * L, L)]
            cond = v > ms[a]
            ms[a] = jnp.maximum(ms[a], v)
            ts[a] = jnp.where(cond, tvec, ts[a])
        tvec = tvec + 1
    m, tw = ms[0], ts[0]
    aw = jnp.zeros((L,), jnp.int32)
    for a in range(1, N_ACC):
        cond = ms[a] > m
        m = jnp.maximum(m, ms[a])
        tw = jnp.where(cond, ts[a], tw)
        aw = jnp.where(cond, jnp.full((L,), a, jnp.int32), aw)
    colv = (aw * CPB + tw) * L + iota16
    rmax = jnp.max(m)
    cand = jnp.where(m == rmax, colv, jnp.full((L,), N_COLS, jnp.int32))
    argc = jnp.min(cand)
    return rmax, argc


def _process_group(buf, g, out_v, orow0, mv_v, uvec, ivec, iota16):
    def row_body(i, carry):
        resm, resc = carry
        for u in range(2):
            r = 2 * i + u
            rmax, argc = _scan_row(buf, g * L + r, iota16)
            lanemask = iota16 == r
            resm = jnp.where(lanemask, rmax, resm)
            resc = jnp.where(lanemask, argc, resc)
        return resm, resc

    resm = jnp.full((L,), -jnp.inf, jnp.float32)
    resc = jnp.zeros((L,), jnp.int32)
    resm, resc = lax.fori_loop(0, L // 2, row_body, (resm, resc))

    below = UNMATCHED_T > resm
    between = jnp.logical_and(resm >= UNMATCHED_T, MATCHED_T > resm)
    c4 = resc * N_OUT
    for j in range(N_OUT):
        gj = plsc.load_gather(mv_v, [c4 + j])
        o = jnp.where(below, uvec, gj)
        o = jnp.where(between, ivec, o)
        out_v[j, pl.ds(orow0, L)] = o


def _body(sim, mv, unm, ign, out, buf0, buf1, mv_v, scal_v, out_v, sem0, sem1):
    c = lax.axis_index("c")
    s = lax.axis_index("s")
    wid = s * NC + c

    pltpu.sync_copy(mv, mv_v)
    pltpu.sync_copy(unm, scal_v.at[pl.ds(0, 1)])
    pltpu.sync_copy(ign, scal_v.at[pl.ds(8, 1)])
    uvec = plsc.load_gather(scal_v, [jnp.zeros((L,), jnp.int32)])
    ivec = plsc.load_gather(scal_v, [jnp.full((L,), 8, jnp.int32)])
    iota16 = lax.iota(jnp.int32, L)

    row0 = wid * SLAB
    nb = jnp.where(wid == NW - 1, TAIL_BLOCKS, FULL_BLOCKS)

    def src(blk):
        return sim.at[pl.ds(row0 + blk * BLOCK_ROWS, BLOCK_ROWS), :]

    def proc_block(buf, blk):
        for g in range(BLOCK_ROWS // L):
            _process_group(buf, g, out_v, blk * BLOCK_ROWS + g * L,
                           mv_v, uvec, ivec, iota16)

    pltpu.async_copy(src(0), buf0, sem0)

    def pair_body(t, carry):
        blk = 2 * t
        pltpu.async_copy(src(blk + 1), buf1, sem1)
        pltpu.make_async_copy(src(blk), buf0, sem0).wait()
        proc_block(buf0, blk)
        pltpu.async_copy(src(blk + 2), buf0, sem0)
        pltpu.make_async_copy(src(blk + 1), buf1, sem1).wait()
        proc_block(buf1, blk + 1)
        return carry

    lax.fori_loop(0, (nb - 1) // 2, pair_body, 0)
    pltpu.make_async_copy(src(nb - 1), buf0, sem0).wait()
    proc_block(buf0, nb - 1)

    pltpu.sync_copy(sim.at[pl.ds(row0 + nb * BLOCK_ROWS, L), :],
                    buf1.at[pl.ds(0, L), :])
    _process_group(buf1, 0, out_v, nb * BLOCK_ROWS, mv_v, uvec, ivec, iota16)

    pltpu.sync_copy(out_v, out.at[:, pl.ds(row0, SLAB)])


_matcher = functools.partial(
    pl.kernel,
    out_type=jax.ShapeDtypeStruct((OUT_ROWS, PAD_COLS), jnp.float32),
    mesh=plsc.VectorSubcoreMesh(core_axis_name="c", subcore_axis_name="s"),
    compiler_params=pltpu.CompilerParams(needs_layout_passes=False),
    scratch_types=[
        pltpu.VMEM((BLOCK_ROWS, N_COLS), jnp.float32),
        pltpu.VMEM((BLOCK_ROWS, N_COLS), jnp.float32),
        pltpu.VMEM((N_COLS * N_OUT,), jnp.float32),
        pltpu.VMEM((L,), jnp.float32),
        pltpu.VMEM((OUT_ROWS, SLAB), jnp.float32),
        pltpu.SemaphoreType.DMA,
        pltpu.SemaphoreType.DMA,
    ],
)(_body)


def kernel(similarity, matched_values, unmatched_values, ignored_values):
    out = _matcher(similarity, matched_values.reshape(-1), unmatched_values,
                   ignored_values)
    return out[:N_OUT, :N_ROWS].T


# speedup vs baseline: 1.3715x; 1.0201x over previous
import functools

import jax
import jax.numpy as jnp
from jax import lax
from jax.experimental import pallas as pl
from jax.experimental.pallas import tpu as pltpu
from jax.experimental.pallas import tpu_sc as plsc

N_ROWS = 20000
N_COLS = 512
N_OUT = 4
NC = 2
NS = 16
NW = NC * NS
L = 16

SLAB = 640
OUT_ROWS = 8
PAD_COLS = NW * SLAB
BLOCK_ROWS = 80
FULL_BLOCKS = 8
TAIL_BLOCKS = 2
N_CHUNKS = N_COLS // L
N_ACC = 4
CPB = N_CHUNKS // N_ACC

MATCHED_T = 0.5
UNMATCHED_T = 0.4


def _scan_row(buf, rr, iota16):
    ninf = jnp.full((L,), -jnp.inf, jnp.float32)
    ms = [ninf for _ in range(N_ACC)]
    ts = [jnp.zeros((L,), jnp.int32) for _ in range(N_ACC)]
    tvec = jnp.zeros((L,), jnp.int32)
    for t in range(CPB):
        for a in range(N_ACC):
            k = a * CPB + t
            v = buf[rr, pl.ds(k * L, L)]
            cond = v > ms[a]
            ms[a] = jnp.maximum(ms[a], v)
            ts[a] = jnp.where(cond, tvec, ts[a])
        tvec = tvec + 1
    m, tw = ms[0], ts[0]
    aw = jnp.zeros((L,), jnp.int32)
    for a in range(1, N_ACC):
        cond = ms[a] > m
        m = jnp.maximum(m, ms[a])
        tw = jnp.where(cond, ts[a], tw)
        aw = jnp.where(cond, jnp.full((L,), a, jnp.int32), aw)
    colv = (aw * CPB + tw) * L + iota16
    rmax = jnp.max(m)
    cand = jnp.where(m == rmax, colv, jnp.full((L,), N_COLS, jnp.int32))
    argc = jnp.min(cand)
    return rmax, argc


def _process_group(buf, g, out_v, orow0, mv_v, uvec, ivec, iota16):
    def row_body(i, carry):
        resm, resc = carry
        for u in range(2):
            r = 2 * i + u
            rmax, argc = _scan_row(buf, g * L + r, iota16)
            lanemask = iota16 == r
            resm = jnp.where(lanemask, rmax, resm)
            resc = jnp.where(lanemask, argc, resc)
        return resm, resc

    resm = jnp.full((L,), -jnp.inf, jnp.float32)
    resc = jnp.zeros((L,), jnp.int32)
    resm, resc = lax.fori_loop(0, L // 2, row_body, (resm, resc))

    below = UNMATCHED_T > resm
    between = jnp.logical_and(resm >= UNMATCHED_T, MATCHED_T > resm)
    c4 = resc * N_OUT
    for j in range(N_OUT):
        gj = plsc.load_gather(mv_v, [c4 + j])
        o = jnp.where(below, uvec, gj)
        o = jnp.where(between, ivec, o)
        out_v[j, pl.ds(orow0, L)] = o


def _body(sim, mv, unm, ign, out, buf0, buf1, mv_v, scal_v, out_v, sem0, sem1):
    c = lax.axis_index("c")
    s = lax.axis_index("s")
    wid = s * NC + c

    pltpu.sync_copy(mv, mv_v)
    pltpu.sync_copy(unm, scal_v.at[pl.ds(0, 1)])
    pltpu.sync_copy(ign, scal_v.at[pl.ds(8, 1)])
    uvec = plsc.load_gather(scal_v, [jnp.zeros((L,), jnp.int32)])
    ivec = plsc.load_gather(scal_v, [jnp.full((L,), 8, jnp.int32)])
    iota16 = lax.iota(jnp.int32, L)

    row0 = wid * SLAB
    nb = jnp.where(wid == NW - 1, TAIL_BLOCKS, FULL_BLOCKS)

    def src(blk):
        row = jnp.minimum(row0 + blk * BLOCK_ROWS, N_ROWS - BLOCK_ROWS)
        return sim.at[pl.ds(row, BLOCK_ROWS), :]

    def proc_block(buf, blk):
        for g in range(BLOCK_ROWS // L):
            _process_group(buf, g, out_v, blk * BLOCK_ROWS + g * L,
                           mv_v, uvec, ivec, iota16)

    pltpu.async_copy(src(0), buf0, sem0)

    def pair_body(t, carry):
        blk = 2 * t
        pltpu.async_copy(src(blk + 1), buf1, sem1)
        pltpu.make_async_copy(src(blk), buf0, sem0).wait()
        proc_block(buf0, blk)
        pltpu.async_copy(src(blk + 2), buf0, sem0)
        pltpu.make_async_copy(src(blk + 1), buf1, sem1).wait()
        proc_block(buf1, blk + 1)
        return carry

    lax.fori_loop(0, nb // 2, pair_body, 0)

    pltpu.sync_copy(out_v, out.at[:, pl.ds(row0, SLAB)])


_matcher = functools.partial(
    pl.kernel,
    out_type=jax.ShapeDtypeStruct((OUT_ROWS, PAD_COLS), jnp.float32),
    mesh=plsc.VectorSubcoreMesh(core_axis_name="c", subcore_axis_name="s"),
    compiler_params=pltpu.CompilerParams(needs_layout_passes=False),
    scratch_types=[
        pltpu.VMEM((BLOCK_ROWS, N_COLS), jnp.float32),
        pltpu.VMEM((BLOCK_ROWS, N_COLS), jnp.float32),
        pltpu.VMEM((N_COLS * N_OUT,), jnp.float32),
        pltpu.VMEM((L,), jnp.float32),
        pltpu.VMEM((OUT_ROWS, SLAB), jnp.float32),
        pltpu.SemaphoreType.DMA,
        pltpu.SemaphoreType.DMA,
    ],
)(_body)


def kernel(similarity, matched_values, unmatched_values, ignored_values):
    out = _matcher(similarity, matched_values.reshape(-1), unmatched_values,
                   ignored_values)
    return out[:N_OUT, :N_ROWS].T
